# Initial kernel scaffold; baseline (speedup 1.0000x reference)
import jax
import jax.numpy as jnp
from jax.experimental import pallas as pl


def kernel(x, ring, W, b, gamma, beta, mean, var):
    raise NotImplementedError("write your pallas kernel here")


# speedup vs baseline: 9.2281x; 9.2281x over previous
import functools
import jax
import jax.numpy as jnp
from jax.experimental import pallas as pl
from jax.experimental.pallas import tpu as pltpu

_B, _N = 16, 16384
_DIN, _DOUT = 64, 128
_EPS = 1e-5
_NB1 = 2048
_NB2 = 2048
_NEG = -3.0e38


def _stage1_body(x_ref, ring_ref, a0_ref, a1_ref, out_ref):
    j = pl.program_id(1)

    @pl.when(j == 0)
    def _():
        out_ref[...] = jnp.full_like(out_ref, _NEG)

    xb = x_ref[0]
    r = ring_ref[0, 0]
    h0 = jnp.dot(a0_ref[...], xb, preferred_element_type=jnp.float32)
    h1 = jnp.dot(a1_ref[...], xb, preferred_element_type=jnp.float32)
    is0 = (r == 0)[None, :]
    p0 = jnp.max(jnp.where(is0, h0, _NEG), axis=1)
    p1 = jnp.max(jnp.where(is0, _NEG, h1), axis=1)
    acc = jnp.concatenate([p0[None], p1[None]], axis=0)
    out_ref[0] = jnp.maximum(out_ref[0], acc)


def _stage2_body(ring_ref, m_ref, out_ref):
    r = ring_ref[0, 0]
    m0 = m_ref[0, 0, :]
    m1 = m_ref[0, 1, :]
    is0 = (r == 0)[None, :]
    out_ref[0] = jnp.where(is0, m0[:, None], m1[:, None])


def kernel(x, ring, W, b, gamma, beta, mean, var):
    Bx, Din, Nx = x.shape
    R = W.shape[0]
    Dout = W.shape[1]
    nj1 = Nx // _NB1
    nj2 = Nx // _NB2

    scale = gamma / jnp.sqrt(var + _EPS)
    A = scale[:, :, None] * W
    c = scale * (b - mean) + beta

    ring = ring.astype(jnp.int32)
    ring3a = ring.reshape(Bx * nj1, 1, _NB1)

    m_raw = pl.pallas_call(
        _stage1_body,
        grid=(Bx, nj1),
        in_specs=[
            pl.BlockSpec((1, Din, _NB1), lambda bi, j: (bi, 0, j)),
            pl.BlockSpec((1, 1, _NB1), lambda bi, j, nj=nj1: (bi * nj + j, 0, 0)),
            pl.BlockSpec((Dout, Din), lambda bi, j: (0, 0)),
            pl.BlockSpec((Dout, Din), lambda bi, j: (0, 0)),
        ],
        out_specs=pl.BlockSpec((1, R, Dout), lambda bi, j: (bi, 0, 0)),
        out_shape=jax.ShapeDtypeStruct((Bx, R, Dout), jnp.float32),
        compiler_params=pltpu.CompilerParams(
            dimension_semantics=("arbitrary", "arbitrary"),
        ),
    )(x, ring3a, A[0], A[1])

    M = m_raw + c[None, :, :]

    ring3b = ring.reshape(Bx * nj2, 1, _NB2)
    out = pl.pallas_call(
        _stage2_body,
        grid=(Bx, nj2),
        in_specs=[
            pl.BlockSpec((1, 1, _NB2), lambda bi, j, nj=nj2: (bi * nj + j, 0, 0)),
            pl.BlockSpec((1, R, Dout), lambda bi, j: (bi, 0, 0)),
        ],
        out_specs=pl.BlockSpec((1, Dout, _NB2), lambda bi, j: (bi, 0, j)),
        out_shape=jax.ShapeDtypeStruct((Bx, Dout, Nx), jnp.float32),
        compiler_params=pltpu.CompilerParams(
            dimension_semantics=("arbitrary", "arbitrary"),
        ),
    )(ring3b, M)
    return out


# speedup vs baseline: 18.5423x; 2.0093x over previous
import functools
import jax
import jax.numpy as jnp
from jax.experimental import pallas as pl
from jax.experimental.pallas import tpu as pltpu

_EPS = 1e-5
_NB1 = 8192
_NB2 = 4096
_NEG = -1.0e9
_KAUG = 72


def _stage1_body(x_ref, ring_ref, aaug_ref, out_ref, acc_ref, *, nb, nj):
    j = pl.program_id(1)

    xb = x_ref[0].astype(jnp.bfloat16)
    r = ring_ref[0, 0]
    madd0 = jnp.where(r == 0, 0.0, _NEG)
    madd1 = jnp.where(r == 0, _NEG, 0.0)
    srow = jax.lax.broadcasted_iota(jnp.int32, (_KAUG - 64, nb), 0)
    mrows = jnp.where(srow == 0, madd0[None, :],
                      jnp.where(srow == 1, madd1[None, :], 0.0))
    xaug = jnp.concatenate([xb, mrows.astype(jnp.bfloat16)], axis=0)
    h = jnp.dot(aaug_ref[...], xaug, preferred_element_type=jnp.float32)
    p = h[:, 0:128]
    for k in range(1, nb // 128):
        p = jnp.maximum(p, h[:, k * 128:(k + 1) * 128])

    @pl.when(j == 0)
    def _():
        acc_ref[...] = p

    @pl.when(j > 0)
    def _():
        acc_ref[...] = jnp.maximum(acc_ref[...], p)

    @pl.when(j == nj - 1)
    def _():
        out_ref[0, 0] = jnp.max(acc_ref[...], axis=1)


def _stage2_body(ring_ref, m_ref, out_ref, *, nb):
    r = ring_ref[0, 0]
    is0f = (r == 0).astype(jnp.float32)[None, :]
    diff = m_ref[0, :, 0:1]
    m1 = m_ref[0, :, 1:2]
    out_ref[0] = m1 + is0f * diff


def kernel(x, ring, W, b, gamma, beta, mean, var):
    Bx, Din, Nx = x.shape
    R = W.shape[0]
    Dout = W.shape[1]
    nb1 = min(_NB1, Nx)
    nb2 = min(_NB2, Nx)
    nj1 = Nx // nb1
    nj2 = Nx // nb2

    scale = gamma / jnp.sqrt(var + _EPS)
    A = scale[:, :, None] * W
    c = scale * (b - mean) + beta

    Acat = A.reshape(R * Dout, Din)
    ekatze = jnp.zeros((R * Dout, _KAUG - Din), jnp.float32)
    ekatze = ekatze.at[:Dout, 0].set(1.0).at[Dout:, 1].set(1.0)
    Aaug = jnp.concatenate([Acat, ekatze], axis=1).astype(jnp.bfloat16)

    ring = ring.astype(jnp.int32)
    ring3a = ring.reshape(Bx * nj1, 1, nb1)

    m_raw = pl.pallas_call(
        functools.partial(_stage1_body, nb=nb1, nj=nj1),
        grid=(Bx, nj1),
        in_specs=[
            pl.BlockSpec((1, Din, nb1), lambda bi, j: (bi, 0, j)),
            pl.BlockSpec((1, 1, nb1), lambda bi, j, nj=nj1: (bi * nj + j, 0, 0)),
            pl.BlockSpec((R * Dout, _KAUG), lambda bi, j: (0, 0)),
        ],
        out_specs=pl.BlockSpec((1, 1, R * Dout), lambda bi, j: (bi, 0, 0)),
        out_shape=jax.ShapeDtypeStruct((Bx, 1, R * Dout), jnp.float32),
        scratch_shapes=[pltpu.VMEM((R * Dout, 128), jnp.float32)],
        compiler_params=pltpu.CompilerParams(
            dimension_semantics=("arbitrary", "arbitrary"),
        ),
    )(x, ring3a, Aaug)

    M = m_raw.reshape(Bx, R, Dout) + c[None, :, :]
    Msel = jnp.stack([M[:, 0, :] - M[:, 1, :], M[:, 1, :]], axis=2)
    Msel = jnp.concatenate(
        [Msel, jnp.zeros((Bx, Dout, 6), jnp.float32)], axis=2)

    ring3b = ring.reshape(Bx * nj2, 1, nb2)
    out = pl.pallas_call(
        functools.partial(_stage2_body, nb=nb2),
        grid=(Bx, nj2),
        in_specs=[
            pl.BlockSpec((1, 1, nb2), lambda bi, j, nj=nj2: (bi * nj + j, 0, 0)),
            pl.BlockSpec((1, Dout, 8), lambda bi, j: (bi, 0, 0)),
        ],
        out_specs=pl.BlockSpec((1, Dout, nb2), lambda bi, j: (bi, 0, j)),
        out_shape=jax.ShapeDtypeStruct((Bx, Dout, Nx), jnp.float32),
        compiler_params=pltpu.CompilerParams(
            dimension_semantics=("arbitrary", "arbitrary"),
        ),
    )(ring3b, Msel)
    return out
